# baseline (speedup 1.0000x reference)
import functools

import jax
import jax.numpy as jnp
from jax import lax
from jax.experimental import pallas as pl
from jax.experimental.pallas import tpu as pltpu
from jax.experimental.pallas import tpu_sc as plsc

_N = 10000
_E = 320000
_NC = 2
_NS = 16
_L = 16
_CH = 80
_T2 = 5.0
_T3 = 0.1

@functools.lru_cache(maxsize=None)
def _edge_kernel(D):
    DE = D + 8
    NW = _NC * _NS
    EPW = _E // NW
    NCHUNK = EPW // _CH
    RSTRIDE = 624
    RROWS = 640
    NSEG = D // _L
    ZOFF = list(range(0, DE - _L + 1, _L))
    if ZOFF[-1] != DE - _L:
        ZOFF.append(DE - _L)

    mesh = plsc.VectorSubcoreMesh(core_axis_name="c", subcore_axis_name="s")

    @functools.partial(
        pl.kernel,
        out_type=jax.ShapeDtypeStruct((_NC * _N, DE), jnp.float32),
        mesh=mesh,
        compiler_params=pltpu.CompilerParams(
            use_tc_tiling_on_sc=False, needs_layout_passes=False),
        scratch_types=[
            pltpu.VMEM((2, _CH), jnp.int32),
            pltpu.VMEM((2, _CH), jnp.int32),
            pltpu.VMEM((_CH, DE), jnp.float32),
            pltpu.VMEM((_CH, DE), jnp.float32),
            pltpu.VMEM((_CH, DE), jnp.float32),
            pltpu.VMEM((_CH, DE), jnp.float32),
            pltpu.VMEM((_CH,), jnp.float32),
            pltpu.VMEM((_L, _L), jnp.float32),
            pltpu.VMEM_SHARED((_N, DE), jnp.float32),
            pltpu.SemaphoreType.DMA, pltpu.SemaphoreType.DMA,
            pltpu.SemaphoreType.DMA, pltpu.SemaphoreType.DMA,
        ],
    )
    def ek(feat_hbm, idx_hbm, num_hbm,
           idxA, idxB, srA, drA, srB, drB, dots, dtmp, num_sh,
           sgA1, sgA2, sgB1, sgB2):
        c = lax.axis_index("c")
        s = lax.axis_index("s")
        w = c * _NS + s
        base = w * NCHUNK

        zero = jnp.zeros((_L,), jnp.float32)
        zero_i = jnp.zeros((_L,), jnp.int32)
        iota16 = lax.iota(jnp.int32, _L)

        def zero_body(i, _):
            for off in ZOFF:
                srA[i, off:off + _L] = zero
            return 0

        lax.fori_loop(0, _CH, zero_body, 0, unroll=4)

        rbase = pl.multiple_of(s * RSTRIDE, 8)
        for k in range(RROWS // _CH):
            pltpu.sync_copy(srA, num_sh.at[pl.ds(rbase + k * _CH, _CH)])
        plsc.subcore_barrier()

        def load_idx(kc, idxX):
            pltpu.sync_copy(idx_hbm.at[pl.ds(2 * (base + kc), 2)], idxX)

        def gather_issue(idxX, sr, dr, g1, g2):
            pltpu.async_copy(feat_hbm.at[idxX.at[0]], sr, g1)
            pltpu.async_copy(feat_hbm.at[idxX.at[1]], dr, g2)

        def compute(idxX, sr, dr, g1, g2):
            pltpu.make_async_copy(feat_hbm.at[idxX.at[0]], sr, g1).wait()
            pltpu.make_async_copy(feat_hbm.at[idxX.at[1]], dr, g2).wait()
            for g in range(_CH // _L):
                def dot_body(i, _):
                    r = g * _L + i
                    acc = sr[r, 0:_L] * dr[r, 0:_L]
                    for k in range(1, NSEG):
                        acc = acc + (sr[r, k * _L:(k + 1) * _L]
                                     * dr[r, k * _L:(k + 1) * _L])
                    dtmp[i] = acc
                    return 0

                lax.fori_loop(0, _L, dot_body, 0, unroll=8)
                dv = plsc.load_gather(dtmp, [iota16, zero_i])
                for j in range(1, _L):
                    dv = dv + plsc.load_gather(
                        dtmp, [iota16, jnp.full((_L,), j, jnp.int32)])
                rows = iota16 + (g * _L)
                colD = jnp.full((_L,), D, jnp.int32)
                nrm = plsc.load_gather(sr, [rows, colD])
                wv = jnp.exp(dv)
                dots[pl.ds(g * _L, _L)] = wv * nrm
                plsc.store_scatter(sr, [rows, colD], wv)

            def scale_body(i, _):
                mv = plsc.load_gather(dots, [jnp.full((_L,), i, jnp.int32)])
                for k in range(NSEG):
                    sr[i, k * _L:(k + 1) * _L] = (
                        sr[i, k * _L:(k + 1) * _L] * mv)
                return 0

            lax.fori_loop(0, _CH, scale_body, 0, unroll=8)

            pltpu.sync_copy(sr, num_sh.at[idxX.at[1]], add=True)

        load_idx(0, idxA)
        gather_issue(idxA, srA, drA, sgA1, sgA2)

        def pair_body(t, _):
            k0 = 2 * t
            load_idx(k0 + 1, idxB)
            gather_issue(idxB, srB, drB, sgB1, sgB2)
            compute(idxA, srA, drA, sgA1, sgA2)
            load_idx(k0 + 2, idxA)
            gather_issue(idxA, srA, drA, sgA1, sgA2)
            compute(idxB, srB, drB, sgB1, sgB2)
            return 0

        lax.fori_loop(0, (NCHUNK - 1) // 2, pair_body, 0)
        compute(idxA, srA, drA, sgA1, sgA2)

        plsc.subcore_barrier()
        pltpu.sync_copy(num_sh.at[pl.ds(rbase, RROWS)],
                        num_hbm.at[pl.ds(c * _N + rbase, RROWS)])

    return ek


_BROWS = 2000
_NBLK = _N // _BROWS


def _ext_block(f):
    ss = jnp.sum(f * f, axis=1, keepdims=True)
    nrm = jnp.maximum(jnp.sqrt(ss), 1e-12)
    nh = f * (1.0 / nrm)
    pad = jnp.zeros((f.shape[0], 7), jnp.float32)
    return jnp.concatenate([nh, nrm, pad], axis=1)


def _prep_body(x_ref, o_ref):
    o_ref[...] = _ext_block(x_ref[...])


def _prep_tc(x):
    return pl.pallas_call(
        _prep_body,
        grid=(_NBLK,),
        in_specs=[pl.BlockSpec((_BROWS, 128), lambda i: (i, 0))],
        out_specs=pl.BlockSpec((_BROWS, 136), lambda i: (i, 0)),
        out_shape=jax.ShapeDtypeStruct((_N, 136), jnp.float32),
    )(x)


def _dual_specs(DE):
    return [
        pl.BlockSpec((_BROWS, DE), lambda i: (i, 0)),
        pl.BlockSpec((_BROWS, DE), lambda i: (i + _NBLK, 0)),
    ]


def _combine_blocks(a_ref, b_ref, D):
    num = a_ref[...] + b_ref[...]
    den = jnp.maximum(num[:, D:D + 1], 1e-12)
    return num[:, 0:D] / den


@functools.lru_cache(maxsize=None)
def _combine_ext_tc(D):
    DE = D + 8

    def body(a_ref, b_ref, ext_ref, fea_ref):
        fea = _combine_blocks(a_ref, b_ref, D)
        fea_ref[...] = fea
        ext_ref[...] = _ext_block(fea)

    call = pl.pallas_call(
        body,
        grid=(_NBLK,),
        in_specs=_dual_specs(DE),
        out_specs=(pl.BlockSpec((_BROWS, DE), lambda i: (i, 0)),
                   pl.BlockSpec((_BROWS, D), lambda i: (i, 0))),
        out_shape=(jax.ShapeDtypeStruct((_N, DE), jnp.float32),
                   jax.ShapeDtypeStruct((_N, D), jnp.float32)),
    )
    return lambda acc: call(acc, acc)


def _middle_body(fea2_ref, mw_ref, g_ref, b_ref, rc_ref,
                 fea_ref, f_ref, p_ref, ext_ref):
    fea = fea2_ref[...] * mw_ref[...]
    fea_ref[...] = fea
    f = fea[:, 0:16]
    for m in range(1, 8):
        f = f + fea[:, m * 16:(m + 1) * 16]
    mean = jnp.mean(f, axis=0, keepdims=True)
    var = jnp.mean((f - mean) ** 2, axis=0, keepdims=True)
    fb = (f - mean) / jnp.sqrt(var + 1e-5) * g_ref[...] + b_ref[...]
    fb = jnp.where(fb > 0, fb, 0.25 * fb)
    f_ref[...] = fb
    z = fb * (1.0 / _T3)
    z = z - jnp.max(z, axis=1, keepdims=True)
    ez = jnp.exp(z)
    sm = ez / jnp.sum(ez, axis=1, keepdims=True)
    p_ref[...] = jnp.dot(sm, rc_ref[...], preferred_element_type=jnp.float32)
    ext_ref[...] = _ext_block(fb)


def _middle_tc(fea2, mw, g, b, rc):
    return pl.pallas_call(
        _middle_body,
        out_shape=(jax.ShapeDtypeStruct((_N, 128), jnp.float32),
                   jax.ShapeDtypeStruct((_N, 16), jnp.float32),
                   jax.ShapeDtypeStruct((_N, 16), jnp.float32),
                   jax.ShapeDtypeStruct((_N, 24), jnp.float32)),
    )(fea2, mw, g, b, rc)


def _final_body(a_ref, b_ref, o_ref):
    out4 = _combine_blocks(a_ref, b_ref, 16)
    z = out4 * (1.0 / _T2)
    z = z - jnp.max(z, axis=1, keepdims=True)
    ez = jnp.exp(z)
    o_ref[...] = ez / jnp.sum(ez, axis=1, keepdims=True)


def _final_tc(acc):
    call = pl.pallas_call(
        _final_body,
        grid=(_NBLK,),
        in_specs=_dual_specs(24),
        out_specs=pl.BlockSpec((_BROWS, 16), lambda i: (i, 0)),
        out_shape=jax.ShapeDtypeStruct((_N, 16), jnp.float32),
    )
    return call(acc, acc)


def kernel(x, edge_index, marker_weight, bn_gamma, bn_beta, ref_center):
    nchunk = _E // (_NC * _NS) // _CH
    src2 = edge_index[0].reshape(_NC * _NS, nchunk, _CH)
    dst2 = edge_index[1].reshape(_NC * _NS, nchunk, _CH)
    idx3 = jnp.stack([src2, dst2], axis=2).reshape(-1, _CH)

    ext1 = _prep_tc(x)
    acc1 = _edge_kernel(128)(ext1, idx3)
    ext2, _ = _combine_ext_tc(128)(acc1)
    acc2 = _edge_kernel(128)(ext2, idx3)
    _, fea2 = _combine_ext_tc(128)(acc2)
    fea, f, p, ext3 = _middle_tc(
        fea2,
        marker_weight.reshape(1, 128),
        bn_gamma.reshape(1, 16),
        bn_beta.reshape(1, 16),
        ref_center,
    )
    acc3 = _edge_kernel(16)(ext3, idx3)
    ext4, _ = _combine_ext_tc(16)(acc3)
    acc4 = _edge_kernel(16)(ext4, idx3)
    out = _final_tc(acc4)
    return fea, f, p, out


# speedup vs baseline: 1.1247x; 1.1247x over previous
import functools

import jax
import jax.numpy as jnp
from jax import lax
from jax.experimental import pallas as pl
from jax.experimental.pallas import tpu as pltpu
from jax.experimental.pallas import tpu_sc as plsc

_N = 10000
_E = 320000
_NC = 2
_NS = 16
_L = 16
_CH = 80
_T2 = 5.0
_T3 = 0.1

@functools.lru_cache(maxsize=None)
def _edge_kernel(D):
    DE = D + 8
    NW = _NC * _NS
    EPW = _E // NW
    NCHUNK = EPW // _CH
    RSTRIDE = 624
    RROWS = 640
    NSEG = D // _L
    ZOFF = list(range(0, DE - _L + 1, _L))
    if ZOFF[-1] != DE - _L:
        ZOFF.append(DE - _L)

    mesh = plsc.VectorSubcoreMesh(core_axis_name="c", subcore_axis_name="s")

    @functools.partial(
        pl.kernel,
        out_type=jax.ShapeDtypeStruct((_NC * _N, DE), jnp.float32),
        mesh=mesh,
        compiler_params=pltpu.CompilerParams(
            use_tc_tiling_on_sc=False, needs_layout_passes=False),
        scratch_types=[
            pltpu.VMEM((2, _CH), jnp.int32),
            pltpu.VMEM((2, _CH), jnp.int32),
            pltpu.VMEM((_CH, DE), jnp.float32),
            pltpu.VMEM((_CH, DE), jnp.float32),
            pltpu.VMEM((_CH, DE), jnp.float32),
            pltpu.VMEM((_CH, DE), jnp.float32),
            pltpu.VMEM((_L, _L), jnp.float32),
            pltpu.VMEM_SHARED((_N, DE), jnp.float32),
            pltpu.SemaphoreType.DMA, pltpu.SemaphoreType.DMA,
            pltpu.SemaphoreType.DMA, pltpu.SemaphoreType.DMA,
        ],
    )
    def ek(feat_hbm, idx_hbm, num_hbm,
           idxA, idxB, srA, drA, srB, drB, dtmp, num_sh,
           sgA1, sgA2, sgB1, sgB2):
        c = lax.axis_index("c")
        s = lax.axis_index("s")
        w = c * _NS + s
        base = w * NCHUNK

        zero = jnp.zeros((_L,), jnp.float32)
        zero_i = jnp.zeros((_L,), jnp.int32)
        iota16 = lax.iota(jnp.int32, _L)

        def zero_body(i, _):
            for off in ZOFF:
                srA[i, off:off + _L] = zero
            return 0

        lax.fori_loop(0, _CH, zero_body, 0, unroll=4)

        rbase = pl.multiple_of(s * RSTRIDE, 8)
        for k in range(RROWS // _CH):
            pltpu.sync_copy(srA, num_sh.at[pl.ds(rbase + k * _CH, _CH)])
        plsc.subcore_barrier()

        def load_idx(kc, idxX):
            pltpu.sync_copy(idx_hbm.at[pl.ds(2 * (base + kc), 2)], idxX)

        def gather_issue(idxX, sr, dr, g1, g2):
            pltpu.async_copy(feat_hbm.at[idxX.at[0]], sr, g1)
            pltpu.async_copy(feat_hbm.at[idxX.at[1]], dr, g2)

        def compute(idxX, sr, dr, g1, g2):
            pltpu.make_async_copy(feat_hbm.at[idxX.at[0]], sr, g1).wait()
            pltpu.make_async_copy(feat_hbm.at[idxX.at[1]], dr, g2).wait()
            for g in range(_CH // _L):
                def dot_body(i, _):
                    r = g * _L + i
                    acc = sr[r, 0:_L] * dr[r, 0:_L]
                    for k in range(1, NSEG):
                        acc = acc + (sr[r, k * _L:(k + 1) * _L]
                                     * dr[r, k * _L:(k + 1) * _L])
                    dtmp[i] = acc
                    return 0

                lax.fori_loop(0, _L, dot_body, 0, unroll=4)
                cols = [plsc.load_gather(
                            dtmp, [iota16, jnp.full((_L,), j, jnp.int32)])
                        for j in range(_L)]
                while len(cols) > 1:
                    cols = [cols[j] + cols[j + 1]
                            for j in range(0, len(cols), 2)]
                rows = iota16 + (g * _L)
                colD = jnp.full((_L,), D, jnp.int32)
                nrm = plsc.load_gather(sr, [rows, colD])
                wv = jnp.exp(cols[0])
                mv16 = wv * nrm
                plsc.store_scatter(sr, [rows, colD], wv)

                def scale_body(i, _):
                    r = g * _L + i
                    mv = lax.gather(
                        mv16, (zero_i + i)[:, None],
                        dimension_numbers=lax.GatherDimensionNumbers(
                            offset_dims=(), collapsed_slice_dims=(0,),
                            start_index_map=(0,)),
                        slice_sizes=(1,),
                        mode=lax.GatherScatterMode.PROMISE_IN_BOUNDS)
                    for k in range(NSEG):
                        sr[r, k * _L:(k + 1) * _L] = (
                            sr[r, k * _L:(k + 1) * _L] * mv)
                    return 0

                lax.fori_loop(0, _L, scale_body, 0, unroll=4)

            pltpu.sync_copy(sr, num_sh.at[idxX.at[1]], add=True)

        load_idx(0, idxA)
        gather_issue(idxA, srA, drA, sgA1, sgA2)

        def pair_body(t, _):
            k0 = 2 * t
            load_idx(k0 + 1, idxB)
            gather_issue(idxB, srB, drB, sgB1, sgB2)
            compute(idxA, srA, drA, sgA1, sgA2)
            load_idx(k0 + 2, idxA)
            gather_issue(idxA, srA, drA, sgA1, sgA2)
            compute(idxB, srB, drB, sgB1, sgB2)
            return 0

        lax.fori_loop(0, (NCHUNK - 1) // 2, pair_body, 0)
        compute(idxA, srA, drA, sgA1, sgA2)

        plsc.subcore_barrier()
        pltpu.sync_copy(num_sh.at[pl.ds(rbase, RROWS)],
                        num_hbm.at[pl.ds(c * _N + rbase, RROWS)])

    return ek


_BROWS = 2000
_NBLK = _N // _BROWS


def _ext_block(f):
    ss = jnp.sum(f * f, axis=1, keepdims=True)
    nrm = jnp.maximum(jnp.sqrt(ss), 1e-12)
    nh = f * (1.0 / nrm)
    pad = jnp.zeros((f.shape[0], 7), jnp.float32)
    return jnp.concatenate([nh, nrm, pad], axis=1)


def _prep_body(x_ref, o_ref):
    o_ref[...] = _ext_block(x_ref[...])


def _prep_tc(x):
    return pl.pallas_call(
        _prep_body,
        grid=(_NBLK,),
        in_specs=[pl.BlockSpec((_BROWS, 128), lambda i: (i, 0))],
        out_specs=pl.BlockSpec((_BROWS, 136), lambda i: (i, 0)),
        out_shape=jax.ShapeDtypeStruct((_N, 136), jnp.float32),
    )(x)


def _dual_specs(DE):
    return [
        pl.BlockSpec((_BROWS, DE), lambda i: (i, 0)),
        pl.BlockSpec((_BROWS, DE), lambda i: (i + _NBLK, 0)),
    ]


def _combine_blocks(a_ref, b_ref, D):
    num = a_ref[...] + b_ref[...]
    den = jnp.maximum(num[:, D:D + 1], 1e-12)
    return num[:, 0:D] / den


@functools.lru_cache(maxsize=None)
def _combine_ext_tc(D):
    DE = D + 8

    def body(a_ref, b_ref, ext_ref, fea_ref):
        fea = _combine_blocks(a_ref, b_ref, D)
        fea_ref[...] = fea
        ext_ref[...] = _ext_block(fea)

    call = pl.pallas_call(
        body,
        grid=(_NBLK,),
        in_specs=_dual_specs(DE),
        out_specs=(pl.BlockSpec((_BROWS, DE), lambda i: (i, 0)),
                   pl.BlockSpec((_BROWS, D), lambda i: (i, 0))),
        out_shape=(jax.ShapeDtypeStruct((_N, DE), jnp.float32),
                   jax.ShapeDtypeStruct((_N, D), jnp.float32)),
    )
    return lambda acc: call(acc, acc)


def _middle_body(fea2_ref, mw_ref, g_ref, b_ref, rc_ref,
                 fea_ref, f_ref, p_ref, ext_ref):
    fea = fea2_ref[...] * mw_ref[...]
    fea_ref[...] = fea
    f = fea[:, 0:16]
    for m in range(1, 8):
        f = f + fea[:, m * 16:(m + 1) * 16]
    mean = jnp.mean(f, axis=0, keepdims=True)
    var = jnp.mean((f - mean) ** 2, axis=0, keepdims=True)
    fb = (f - mean) / jnp.sqrt(var + 1e-5) * g_ref[...] + b_ref[...]
    fb = jnp.where(fb > 0, fb, 0.25 * fb)
    f_ref[...] = fb
    z = fb * (1.0 / _T3)
    z = z - jnp.max(z, axis=1, keepdims=True)
    ez = jnp.exp(z)
    sm = ez / jnp.sum(ez, axis=1, keepdims=True)
    p_ref[...] = jnp.dot(sm, rc_ref[...], preferred_element_type=jnp.float32)
    ext_ref[...] = _ext_block(fb)


def _middle_tc(fea2, mw, g, b, rc):
    return pl.pallas_call(
        _middle_body,
        out_shape=(jax.ShapeDtypeStruct((_N, 128), jnp.float32),
                   jax.ShapeDtypeStruct((_N, 16), jnp.float32),
                   jax.ShapeDtypeStruct((_N, 16), jnp.float32),
                   jax.ShapeDtypeStruct((_N, 24), jnp.float32)),
    )(fea2, mw, g, b, rc)


def _final_body(a_ref, b_ref, o_ref):
    out4 = _combine_blocks(a_ref, b_ref, 16)
    z = out4 * (1.0 / _T2)
    z = z - jnp.max(z, axis=1, keepdims=True)
    ez = jnp.exp(z)
    o_ref[...] = ez / jnp.sum(ez, axis=1, keepdims=True)


def _final_tc(acc):
    call = pl.pallas_call(
        _final_body,
        grid=(_NBLK,),
        in_specs=_dual_specs(24),
        out_specs=pl.BlockSpec((_BROWS, 16), lambda i: (i, 0)),
        out_shape=jax.ShapeDtypeStruct((_N, 16), jnp.float32),
    )
    return call(acc, acc)


def kernel(x, edge_index, marker_weight, bn_gamma, bn_beta, ref_center):
    nchunk = _E // (_NC * _NS) // _CH
    src2 = edge_index[0].reshape(_NC * _NS, nchunk, _CH)
    dst2 = edge_index[1].reshape(_NC * _NS, nchunk, _CH)
    idx3 = jnp.stack([src2, dst2], axis=2).reshape(-1, _CH)

    ext1 = _prep_tc(x)
    acc1 = _edge_kernel(128)(ext1, idx3)
    ext2, _ = _combine_ext_tc(128)(acc1)
    acc2 = _edge_kernel(128)(ext2, idx3)
    _, fea2 = _combine_ext_tc(128)(acc2)
    fea, f, p, ext3 = _middle_tc(
        fea2,
        marker_weight.reshape(1, 128),
        bn_gamma.reshape(1, 16),
        bn_beta.reshape(1, 16),
        ref_center,
    )
    acc3 = _edge_kernel(16)(ext3, idx3)
    ext4, _ = _combine_ext_tc(16)(acc3)
    acc4 = _edge_kernel(16)(ext4, idx3)
    out = _final_tc(acc4)
    return fea, f, p, out
